# baseline (speedup 1.0000x reference)
import functools
import math

import jax
import jax.numpy as jnp
from jax import lax
from jax.experimental import pallas as pl
from jax.experimental.pallas import tpu as pltpu
from jax.experimental.pallas import tpu_sc as plsc

_L = 16
_F = 2
_T = 1 << 19
_BASE = 16
_GROWTH = 2.0
_P1 = -1640531535
_P2 = 805459861

_NC = 2
_NS = 16
_NW = _NC * _NS
_LANES = 16
_CHUNK = 1024
_GROUPS = _CHUNK // _LANES
_OUTW = _L * _F


def _level_params():
    params = []
    for l in range(_L):
        scale = _BASE * (_GROWTH ** l) - 1.0
        resolution = int(math.ceil(scale)) + 1
        stride = resolution + 1
        dense = stride ** 3 <= _T
        params.append((float(scale), stride, dense, l * _T))
    return params


@functools.partial(jax.jit, static_argnums=(4,))
def _run(xs, ys, zs, tbl, n_pts):
    per_worker = n_pts // _NW
    n_chunks = per_worker // _CHUNK
    levels = _level_params()
    mesh = plsc.VectorSubcoreMesh(core_axis_name="c", subcore_axis_name="s")

    @functools.partial(
        pl.kernel,
        mesh=mesh,
        out_type=jax.ShapeDtypeStruct((n_pts // _CHUNK, _OUTW, _CHUNK),
                                      jnp.float32),
        scratch_types=[
            pltpu.VMEM((_CHUNK,), jnp.float32),
            pltpu.VMEM((_CHUNK,), jnp.float32),
            pltpu.VMEM((_CHUNK,), jnp.float32),
            pltpu.VMEM((16 * _CHUNK,), jnp.int32),
            pltpu.VMEM((8 * _CHUNK,), jnp.float32),
            pltpu.VMEM((16 * _CHUNK,), jnp.float32),
            pltpu.VMEM((_OUTW, _CHUNK), jnp.float32),
            pltpu.SemaphoreType.DMA,
        ],
    )
    def hash_encode(xs_hbm, ys_hbm, zs_hbm, tbl_hbm, out_hbm,
                    xs_v, ys_v, zs_v, idx_v, w_v, rows_v, out_v, sem):
        wid = lax.axis_index("s") * _NC + lax.axis_index("c")

        def do_chunk(ci, carry):
            gchunk = wid * n_chunks + ci
            gbase = gchunk * _CHUNK
            pltpu.sync_copy(xs_hbm.at[pl.ds(gbase, _CHUNK)], xs_v)
            pltpu.sync_copy(ys_hbm.at[pl.ds(gbase, _CHUNK)], ys_v)
            pltpu.sync_copy(zs_hbm.at[pl.ds(gbase, _CHUNK)], zs_v)

            for li, (scale, stride, dense, offset) in enumerate(levels):

                def pass_a(g, c, scale=scale, stride=stride, dense=dense,
                           offset=offset):
                    s = g * _LANES
                    x = xs_v[pl.ds(s, _LANES)]
                    y = ys_v[pl.ds(s, _LANES)]
                    z = zs_v[pl.ds(s, _LANES)]
                    px = x * scale + 0.5
                    py = y * scale + 0.5
                    pz = z * scale + 0.5
                    ix = px.astype(jnp.int32)
                    iy = py.astype(jnp.int32)
                    iz = pz.astype(jnp.int32)
                    fx = px - ix.astype(jnp.float32)
                    fy = py - iy.astype(jnp.float32)
                    fz = pz - iz.astype(jnp.float32)
                    if dense:
                        tx0 = ix + offset
                        ty0 = iy * stride
                        tz0 = iz * (stride * stride)
                        tx1 = tx0 + 1
                        ty1 = ty0 + stride
                        tz1 = tz0 + stride * stride
                    else:
                        tx0 = ix
                        tx1 = ix + 1
                        ty0 = iy * _P1
                        ty1 = ty0 + _P1
                        tz0 = iz * _P2
                        tz1 = tz0 + _P2
                    txs = (tx0, tx1)
                    tys = (ty0, ty1)
                    tzs = (tz0, tz1)
                    wxs = (1.0 - fx, fx)
                    wys = (1.0 - fy, fy)
                    wzs = (1.0 - fz, fz)
                    wxy = [wxs[cx] * wys[cy] for cy in (0, 1) for cx in (0, 1)]
                    for corner in range(8):
                        cx = corner & 1
                        cy = (corner >> 1) & 1
                        cz = (corner >> 2) & 1
                        if dense:
                            t = txs[cx] + tys[cy] + tzs[cz]
                        else:
                            t = ((txs[cx] ^ tys[cy] ^ tzs[cz]) & (_T - 1)) + offset
                        t2 = t + t
                        w = wxy[cy * 2 + cx] * wzs[cz]
                        idx_v[pl.ds((2 * corner) * _CHUNK + s, _LANES)] = t2
                        idx_v[pl.ds((2 * corner + 1) * _CHUNK + s, _LANES)] = t2 + 1
                        w_v[pl.ds(corner * _CHUNK + s, _LANES)] = w
                    return c

                lax.fori_loop(0, _GROUPS, pass_a, 0)

                pltpu.async_copy(tbl_hbm.at[idx_v], rows_v, sem).wait()

                def pass_b(g, c, li=li):
                    s = g * _LANES
                    o0 = jnp.zeros((_LANES,), jnp.float32)
                    o1 = jnp.zeros((_LANES,), jnp.float32)
                    for corner in range(8):
                        w = w_v[pl.ds(corner * _CHUNK + s, _LANES)]
                        r0 = rows_v[pl.ds((2 * corner) * _CHUNK + s, _LANES)]
                        r1 = rows_v[pl.ds((2 * corner + 1) * _CHUNK + s, _LANES)]
                        o0 = o0 + w * r0
                        o1 = o1 + w * r1
                    out_v[2 * li, pl.ds(s, _LANES)] = o0
                    out_v[2 * li + 1, pl.ds(s, _LANES)] = o1
                    return c

                lax.fori_loop(0, _GROUPS, pass_b, 0)

            pltpu.sync_copy(out_v, out_hbm.at[gchunk])
            return carry

        lax.fori_loop(0, n_chunks, do_chunk, 0)

    return hash_encode(xs, ys, zs, tbl)


def _transpose_blocks(blocks):
    n_chunks = blocks.shape[0]

    def body(in_ref, out_ref):
        out_ref[...] = in_ref[0].T

    return pl.pallas_call(
        body,
        grid=(n_chunks,),
        in_specs=[pl.BlockSpec((1, _OUTW, _CHUNK), lambda i: (i, 0, 0))],
        out_specs=pl.BlockSpec((_CHUNK, _OUTW), lambda i: (i, 0)),
        out_shape=jax.ShapeDtypeStruct((n_chunks * _CHUNK, _OUTW),
                                       jnp.float32),
    )(blocks)


def kernel(inputs, tables, size=1):
    prefix_shape = inputs.shape[:-1]
    x = inputs.reshape(-1, 3)
    n_pts = x.shape[0]
    size_f = jnp.asarray(size, dtype=jnp.float32)
    normalized = jnp.clip((x + size_f) / (2.0 * size_f), 0.0, 1.0)
    nt = normalized.T
    tbl = tables.reshape(_L * _T * _F)
    out = _run(nt[0], nt[1], nt[2], tbl, n_pts)
    out = _transpose_blocks(out)
    return out.reshape(prefix_shape + (_L * _F,))


# speedup vs baseline: 3.2086x; 3.2086x over previous
import functools
import math

import jax
import jax.numpy as jnp
from jax import lax
from jax.experimental import pallas as pl
from jax.experimental.pallas import tpu as pltpu
from jax.experimental.pallas import tpu_sc as plsc

_L = 16
_F = 2
_T = 1 << 19
_BASE = 16
_GROWTH = 2.0
_P1 = -1640531535
_P2 = 805459861

_NC = 2
_NS = 16
_NW = _NC * _NS
_LANES = 16
_CHUNK = 1024
_GROUPS = _CHUNK // _LANES
_OUTW = _L * _F


def _level_params():
    params = []
    for l in range(_L):
        scale = _BASE * (_GROWTH ** l) - 1.0
        resolution = int(math.ceil(scale)) + 1
        stride = resolution + 1
        dense = stride ** 3 <= _T
        params.append((float(scale), stride, dense, l * _T))
    return params


@functools.partial(jax.jit, static_argnums=(4,))
def _run(xs, ys, zs, tbl, n_pts):
    per_worker = n_pts // _NW
    n_chunks = per_worker // _CHUNK
    levels = _level_params()
    mesh = plsc.VectorSubcoreMesh(core_axis_name="c", subcore_axis_name="s")

    @functools.partial(
        pl.kernel,
        mesh=mesh,
        out_type=jax.ShapeDtypeStruct((n_pts * _OUTW,), jnp.float32),
        scratch_types=[
            pltpu.VMEM((_CHUNK,), jnp.float32),
            pltpu.VMEM((_CHUNK,), jnp.float32),
            pltpu.VMEM((_CHUNK,), jnp.float32),
            pltpu.VMEM((16 * _CHUNK,), jnp.int32),
            pltpu.VMEM((16 * _CHUNK,), jnp.int32),
            pltpu.VMEM((8 * _CHUNK,), jnp.float32),
            pltpu.VMEM((8 * _CHUNK,), jnp.float32),
            pltpu.VMEM((16 * _CHUNK,), jnp.float32),
            pltpu.VMEM((16 * _CHUNK,), jnp.float32),
            pltpu.VMEM((4, 8 * _CHUNK), jnp.float32),
            pltpu.SemaphoreType.DMA,
            pltpu.SemaphoreType.DMA,
            pltpu.SemaphoreType.DMA,
        ],
    )
    def hash_encode(xs_hbm, ys_hbm, zs_hbm, tbl_hbm, out_hbm,
                    xs_v, ys_v, zs_v, idx0_v, idx1_v, w0_v, w1_v,
                    rows0_v, rows1_v, out_v, gsem0, gsem1, osem):
        wid = lax.axis_index("s") * _NC + lax.axis_index("c")
        gsems = (gsem0, gsem1)
        idxs = (idx0_v, idx1_v)
        ws = (w0_v, w1_v)
        rows = (rows0_v, rows1_v)

        def do_chunk(ci, carry):
            gchunk = wid * n_chunks + ci
            gbase = gchunk * _CHUNK
            pltpu.sync_copy(xs_hbm.at[pl.ds(gbase, _CHUNK)], xs_v)
            pltpu.sync_copy(ys_hbm.at[pl.ds(gbase, _CHUNK)], ys_v)
            pltpu.sync_copy(zs_hbm.at[pl.ds(gbase, _CHUNK)], zs_v)

            def pass_a(li):
                scale, stride, dense, offset = levels[li]
                idx_v = idxs[li & 1]
                w_v = ws[li & 1]

                def body(g, c):
                    s = g * _LANES
                    x = xs_v[pl.ds(s, _LANES)]
                    y = ys_v[pl.ds(s, _LANES)]
                    z = zs_v[pl.ds(s, _LANES)]
                    px = x * scale + 0.5
                    py = y * scale + 0.5
                    pz = z * scale + 0.5
                    ix = px.astype(jnp.int32)
                    iy = py.astype(jnp.int32)
                    iz = pz.astype(jnp.int32)
                    fx = px - ix.astype(jnp.float32)
                    fy = py - iy.astype(jnp.float32)
                    fz = pz - iz.astype(jnp.float32)
                    if dense:
                        tx0 = ix
                        ty0 = iy * stride
                        tz0 = iz * (stride * stride)
                        tx1 = tx0 + 1
                        ty1 = ty0 + stride
                        tz1 = tz0 + stride * stride
                    else:
                        tx0 = ix
                        tx1 = ix + 1
                        ty0 = iy * _P1
                        ty1 = ty0 + _P1
                        tz0 = iz * _P2
                        tz1 = tz0 + _P2
                    txs = (tx0, tx1)
                    tys = (ty0, ty1)
                    tzs = (tz0, tz1)
                    wxs = (1.0 - fx, fx)
                    wys = (1.0 - fy, fy)
                    wzs = (1.0 - fz, fz)
                    wxy = [wxs[cx] * wys[cy] for cy in (0, 1) for cx in (0, 1)]
                    for corner in range(8):
                        cx = corner & 1
                        cy = (corner >> 1) & 1
                        cz = (corner >> 2) & 1
                        if dense:
                            t = txs[cx] + tys[cy] + tzs[cz]
                        else:
                            t = (txs[cx] ^ tys[cy] ^ tzs[cz]) & (_T - 1)
                        tb = (t + t) - (t & 127) + (offset * _F)
                        w = wxy[cy * 2 + cx] * wzs[cz]
                        idx_v[pl.ds((2 * corner) * _CHUNK + s, _LANES)] = tb
                        idx_v[pl.ds((2 * corner + 1) * _CHUNK + s,
                                    _LANES)] = tb + 128
                        w_v[pl.ds(corner * _CHUNK + s, _LANES)] = w
                    return c

                lax.fori_loop(0, _GROUPS, body, 0)

            def start_gather(li):
                b = li & 1
                return pltpu.async_copy(
                    tbl_hbm.at[idxs[b]], rows[b], gsems[b])

            def pass_b(li):
                w_v = ws[li & 1]
                rows_v = rows[li & 1]

                def body(g, c):
                    s = g * _LANES
                    o0 = jnp.zeros((_LANES,), jnp.float32)
                    o1 = jnp.zeros((_LANES,), jnp.float32)
                    for corner in range(8):
                        w = w_v[pl.ds(corner * _CHUNK + s, _LANES)]
                        r0 = rows_v[pl.ds((2 * corner) * _CHUNK + s,
                                          _LANES)]
                        r1 = rows_v[pl.ds((2 * corner + 1) * _CHUNK + s,
                                          _LANES)]
                        o0 = o0 + w * r0
                        o1 = o1 + w * r1
                    pos0 = (g >> 3) * 1024 + (g & 7) * _LANES + (
                        ((2 * li) & 7) * 128)
                    out_v[li >> 2, pl.ds(pos0, _LANES)] = o0
                    out_v[li >> 2, pl.ds(pos0 + 128, _LANES)] = o1
                    return c

                lax.fori_loop(0, _GROUPS, body, 0)

            def drain_plane(cb):
                return pltpu.async_copy(
                    out_v.at[cb],
                    out_hbm.at[pl.ds(cb * (n_pts * 8) + gchunk * (8 * _CHUNK),
                                     8 * _CHUNK)],
                    osem)

            pass_a(0)
            handles = {0: start_gather(0)}
            drains = []
            for li in range(1, _L):
                pass_a(li)
                handles[li - 1].wait()
                handles[li] = start_gather(li)
                pass_b(li - 1)
                if li & 3 == 0:
                    drains.append(drain_plane((li >> 2) - 1))
            handles[_L - 1].wait()
            pass_b(_L - 1)
            drains.append(drain_plane(3))
            for h in drains:
                h.wait()
            return carry

        lax.fori_loop(0, n_chunks, do_chunk, 0)

    return hash_encode(xs, ys, zs, tbl)


def kernel(inputs, tables, size=1):
    prefix_shape = inputs.shape[:-1]
    x = inputs.reshape(-1, 3)
    n_pts = x.shape[0]
    size_f = jnp.asarray(size, dtype=jnp.float32)
    normalized = jnp.clip((x + size_f) / (2.0 * size_f), 0.0, 1.0)
    nt = normalized.T
    tbl = (tables.reshape(_L, _T // 128, 128, _F)
           .transpose(0, 1, 3, 2)
           .reshape(_L * _T * _F))
    out = _run(nt[0], nt[1], nt[2], tbl, n_pts)
    out = (out.reshape(4, n_pts // 128, 8, 128)
           .transpose(1, 3, 0, 2)
           .reshape(n_pts, _OUTW))
    return out.reshape(prefix_shape + (_L * _F,))


# speedup vs baseline: 3.4300x; 1.0690x over previous
import functools
import math

import jax
import jax.numpy as jnp
from jax import lax
from jax.experimental import pallas as pl
from jax.experimental.pallas import tpu as pltpu
from jax.experimental.pallas import tpu_sc as plsc

_L = 16
_F = 2
_T = 1 << 19
_BASE = 16
_GROWTH = 2.0
_P1 = -1640531535
_P2 = 805459861

_NC = 2
_NS = 16
_NW = _NC * _NS
_LANES = 16
_CHUNK = 512
_GROUPS = _CHUNK // _LANES
_OUTW = _L * _F
_NBUF = 4
_DEPTH = 2


def _level_params():
    params = []
    for l in range(_L):
        scale = _BASE * (_GROWTH ** l) - 1.0
        resolution = int(math.ceil(scale)) + 1
        stride = resolution + 1
        dense = stride ** 3 <= _T
        params.append((float(scale), stride, dense, l * _T))
    return params


@functools.partial(jax.jit, static_argnums=(4,))
def _run(xs, ys, zs, tbl, n_pts):
    per_worker = n_pts // _NW
    n_chunks = per_worker // _CHUNK
    levels = _level_params()
    mesh = plsc.VectorSubcoreMesh(core_axis_name="c", subcore_axis_name="s")

    scratch = [
        pltpu.VMEM((_CHUNK,), jnp.float32),
        pltpu.VMEM((_CHUNK,), jnp.float32),
        pltpu.VMEM((_CHUNK,), jnp.float32),
    ]
    scratch += [pltpu.VMEM((16 * _CHUNK,), jnp.int32)
                for _ in range(_NBUF)]
    scratch += [pltpu.VMEM((8 * _CHUNK,), jnp.float32)
                for _ in range(_NBUF)]
    scratch += [pltpu.VMEM((16 * _CHUNK,), jnp.float32)
                for _ in range(_NBUF)]
    scratch += [pltpu.VMEM((4, 8 * _CHUNK), jnp.float32)]
    scratch += [pltpu.SemaphoreType.DMA for _ in range(_NBUF + 1)]

    @functools.partial(
        pl.kernel,
        mesh=mesh,
        out_type=jax.ShapeDtypeStruct((n_pts * _OUTW,), jnp.float32),
        scratch_types=scratch,
    )
    def hash_encode(xs_hbm, ys_hbm, zs_hbm, tbl_hbm, out_hbm,
                    xs_v, ys_v, zs_v, *bufs):
        idxs = bufs[0:_NBUF]
        ws = bufs[_NBUF:2 * _NBUF]
        rows = bufs[2 * _NBUF:3 * _NBUF]
        out_v = bufs[3 * _NBUF]
        gsems = bufs[3 * _NBUF + 1:3 * _NBUF + 1 + _NBUF]
        osem = bufs[3 * _NBUF + 1 + _NBUF]
        wid = lax.axis_index("s") * _NC + lax.axis_index("c")

        def do_chunk(ci, carry):
            gchunk = wid * n_chunks + ci
            gbase = gchunk * _CHUNK
            pltpu.sync_copy(xs_hbm.at[pl.ds(gbase, _CHUNK)], xs_v)
            pltpu.sync_copy(ys_hbm.at[pl.ds(gbase, _CHUNK)], ys_v)
            pltpu.sync_copy(zs_hbm.at[pl.ds(gbase, _CHUNK)], zs_v)

            def pass_a(li):
                scale, stride, dense, offset = levels[li]
                idx_v = idxs[li % _NBUF]
                w_v = ws[li % _NBUF]

                def body(g, c):
                    s = g * _LANES
                    x = xs_v[pl.ds(s, _LANES)]
                    y = ys_v[pl.ds(s, _LANES)]
                    z = zs_v[pl.ds(s, _LANES)]
                    px = x * scale + 0.5
                    py = y * scale + 0.5
                    pz = z * scale + 0.5
                    ix = px.astype(jnp.int32)
                    iy = py.astype(jnp.int32)
                    iz = pz.astype(jnp.int32)
                    fx = px - ix.astype(jnp.float32)
                    fy = py - iy.astype(jnp.float32)
                    fz = pz - iz.astype(jnp.float32)
                    if dense:
                        tx0 = ix
                        ty0 = iy * stride
                        tz0 = iz * (stride * stride)
                        tx1 = tx0 + 1
                        ty1 = ty0 + stride
                        tz1 = tz0 + stride * stride
                    else:
                        tx0 = ix
                        tx1 = ix + 1
                        ty0 = iy * _P1
                        ty1 = ty0 + _P1
                        tz0 = iz * _P2
                        tz1 = tz0 + _P2
                    txs = (tx0, tx1)
                    tys = (ty0, ty1)
                    tzs = (tz0, tz1)
                    wxs = (1.0 - fx, fx)
                    wys = (1.0 - fy, fy)
                    wzs = (1.0 - fz, fz)
                    wxy = [wxs[cx] * wys[cy] for cy in (0, 1) for cx in (0, 1)]
                    for corner in range(8):
                        cx = corner & 1
                        cy = (corner >> 1) & 1
                        cz = (corner >> 2) & 1
                        if dense:
                            t = txs[cx] + tys[cy] + tzs[cz]
                        else:
                            t = (txs[cx] ^ tys[cy] ^ tzs[cz]) & (_T - 1)
                        tb = (t + t) - (t & 127) + (offset * _F)
                        w = wxy[cy * 2 + cx] * wzs[cz]
                        idx_v[pl.ds((2 * corner) * _CHUNK + s, _LANES)] = tb
                        idx_v[pl.ds((2 * corner + 1) * _CHUNK + s,
                                    _LANES)] = tb + 128
                        w_v[pl.ds(corner * _CHUNK + s, _LANES)] = w
                    return c

                lax.fori_loop(0, _GROUPS, body, 0)

            def start_gather(li):
                b = li % _NBUF
                return pltpu.async_copy(
                    tbl_hbm.at[idxs[b]], rows[b], gsems[b])

            def pass_b(li):
                w_v = ws[li % _NBUF]
                rows_v = rows[li % _NBUF]

                def body(g, c):
                    s = g * _LANES
                    o0 = jnp.zeros((_LANES,), jnp.float32)
                    o1 = jnp.zeros((_LANES,), jnp.float32)
                    for corner in range(8):
                        w = w_v[pl.ds(corner * _CHUNK + s, _LANES)]
                        r0 = rows_v[pl.ds((2 * corner) * _CHUNK + s,
                                          _LANES)]
                        r1 = rows_v[pl.ds((2 * corner + 1) * _CHUNK + s,
                                          _LANES)]
                        o0 = o0 + w * r0
                        o1 = o1 + w * r1
                    pos0 = (g >> 3) * 1024 + (g & 7) * _LANES + (
                        ((2 * li) & 7) * 128)
                    out_v[li >> 2, pl.ds(pos0, _LANES)] = o0
                    out_v[li >> 2, pl.ds(pos0 + 128, _LANES)] = o1
                    return c

                lax.fori_loop(0, _GROUPS, body, 0)

            def drain_plane(cb):
                return pltpu.async_copy(
                    out_v.at[cb],
                    out_hbm.at[pl.ds(cb * (n_pts * 8) + gchunk * (8 * _CHUNK),
                                     8 * _CHUNK)],
                    osem)

            handles = {}
            drains = []
            for li in range(_L + _DEPTH):
                if li < _L:
                    pass_a(li)
                    handles[li] = start_gather(li)
                lb = li - _DEPTH
                if lb >= 0:
                    handles[lb].wait()
                    pass_b(lb)
                    if (lb & 3) == 3:
                        drains.append(drain_plane(lb >> 2))
            for h in drains:
                h.wait()
            return carry

        lax.fori_loop(0, n_chunks, do_chunk, 0)

    return hash_encode(xs, ys, zs, tbl)


def kernel(inputs, tables, size=1):
    prefix_shape = inputs.shape[:-1]
    x = inputs.reshape(-1, 3)
    n_pts = x.shape[0]
    size_f = jnp.asarray(size, dtype=jnp.float32)
    normalized = jnp.clip((x + size_f) / (2.0 * size_f), 0.0, 1.0)
    nt = normalized.T
    tbl = (tables.reshape(_L, _T // 128, 128, _F)
           .transpose(0, 1, 3, 2)
           .reshape(_L * _T * _F))
    out = _run(nt[0], nt[1], nt[2], tbl, n_pts)
    out = (out.reshape(4, n_pts // 128, 8, 128)
           .transpose(1, 3, 0, 2)
           .reshape(n_pts, _OUTW))
    return out.reshape(prefix_shape + (_L * _F,))
